# baseline (speedup 1.0000x reference)
import functools

import jax
import jax.numpy as jnp
from jax import lax
from jax.experimental import pallas as pl
from jax.experimental.pallas import tpu as pltpu
from jax.experimental.pallas import tpu_sc as plsc

_B, _C, _Z, _H, _W = 2, 4, 64, 128, 128
_ROW = _H * _W
_NBZ = _B * _Z
_NELEM = _B * _C * _Z * _H * _W
_INV = 1.0 / _NELEM

_ZB = 8


_LOG2E = 1.4426950408889634
_LN2 = 0.6931471805599453


def _tc_body(pred_ref, gt_ref, w_ref, out_ref):
    g = gt_ref[...]
    acc = jnp.zeros_like(g, dtype=jnp.float32)
    for c in range(_C):
        t = pred_ref[0, c][None] * _LOG2E
        l = jnp.log2(1.0 + jnp.exp2(t))
        acc += jnp.where(g == c, w_ref[c] * (l - t), l)
    part = jnp.sum(acc) * (_LN2 * _INV)

    @pl.when((pl.program_id(0) == 0) & (pl.program_id(1) == 0))
    def _init():
        out_ref[...] = jnp.zeros_like(out_ref)

    out_ref[...] += part


def _kernel_tc(pred, gt, weights):
    grid = (_B, _Z // _ZB)
    out = pl.pallas_call(
        _tc_body,
        grid=grid,
        in_specs=[
            pl.BlockSpec((1, _C, _ZB, _H, _W), lambda b, z: (b, 0, z, 0, 0)),
            pl.BlockSpec((1, _ZB, _H, _W), lambda b, z: (b, z, 0, 0)),
            pl.BlockSpec(memory_space=pltpu.SMEM),
        ],
        out_specs=pl.BlockSpec((1, 1), lambda b, z: (0, 0)),
        out_shape=jax.ShapeDtypeStruct((1, 1), jnp.float32),
    )(pred, gt, weights)
    return out[0, 0]


_NC, _NS = 2, 16
_NW = _NC * _NS
_BZ_PER_W = _NBZ // _NW


def _sc_body(pred_hbm, gt_hbm, w_hbm, out_hbm, pbuf, gbuf, wbuf, obuf,
             psem0, psem1, gsem0, gsem1):
    wid = lax.axis_index("c") * _NS + lax.axis_index("s")
    pltpu.sync_copy(w_hbm, wbuf)
    wvec = wbuf[...]
    psem = (psem0, psem1)
    gsem = (gsem0, gsem1)

    def row_idx(jj, c):
        j = wid * _BZ_PER_W + jj
        b = j // _Z
        return j, (b * _C + c) * _Z + (j - b * _Z)

    nrow = _BZ_PER_W * _C
    pcopy = [None] * nrow
    gcopy = [None] * _BZ_PER_W
    j0, r0 = row_idx(0, 0)
    gcopy[0] = pltpu.async_copy(gt_hbm.at[j0], gbuf.at[0], gsem[0])
    pcopy[0] = pltpu.async_copy(pred_hbm.at[r0], pbuf.at[0], psem[0])

    acc = jnp.zeros((16,), jnp.float32)
    for k in range(nrow):
        jj, c = divmod(k, _C)
        if k + 1 < nrow:
            jj2, c2 = divmod(k + 1, _C)
            j2, r2 = row_idx(jj2, c2)
            pcopy[k + 1] = pltpu.async_copy(
                pred_hbm.at[r2], pbuf.at[(k + 1) % 2], psem[(k + 1) % 2])
            if jj2 != jj:
                gcopy[jj2] = pltpu.async_copy(
                    gt_hbm.at[j2], gbuf.at[jj2 % 2], gsem[jj2 % 2])
        pcopy[k].wait()
        if c == 0:
            gcopy[jj].wait()
        pwc = wvec[c]
        pb = pbuf.at[k % 2]
        gb = gbuf.at[jj % 2]

        def body(i, a, c=c, pwc=pwc, pb=pb, gb=gb):
            x = pb[pl.ds(i * 16, 16)]
            g = gb[pl.ds(i * 16, 16)]
            e = jnp.exp(-jnp.abs(x))
            s = e / (e + 2.0)
            s2 = s * s
            l = s * (2.0 + s2 * (2.0 / 3.0 + s2 * 0.4))
            sp = jnp.maximum(x, 0.0) + l
            return a + jnp.where(g == c, pwc * (sp - x), sp)

        acc = lax.fori_loop(0, _ROW // 16, body, acc, unroll=16)
    obuf[...] = acc * _INV
    pltpu.sync_copy(obuf, out_hbm.at[wid])


def _kernel_sc(pred, gt, weights):
    p2 = pred.reshape(_B * _C * _Z, _ROW)
    g2 = gt.reshape(_NBZ, _ROW)
    wpad = jnp.pad(weights, (0, 16 - _C))
    mesh = plsc.VectorSubcoreMesh(core_axis_name="c", subcore_axis_name="s")
    call = pl.kernel(
        _sc_body,
        out_type=jax.ShapeDtypeStruct((_NW, 16), jnp.float32),
        mesh=mesh,
        scratch_types=[
            pltpu.VMEM((2, _ROW), jnp.float32),
            pltpu.VMEM((2, _ROW), jnp.int32),
            pltpu.VMEM((16,), jnp.float32),
            pltpu.VMEM((16,), jnp.float32),
            pltpu.SemaphoreType.DMA,
            pltpu.SemaphoreType.DMA,
            pltpu.SemaphoreType.DMA,
            pltpu.SemaphoreType.DMA,
        ],
    )
    return jnp.sum(call(p2, g2, wpad))


def kernel(pred, gt, weights):
    return _kernel_sc(pred, gt, weights)


# speedup vs baseline: 2.1310x; 2.1310x over previous
import functools

import jax
import jax.numpy as jnp
from jax import lax
from jax.experimental import pallas as pl
from jax.experimental.pallas import tpu as pltpu
from jax.experimental.pallas import tpu_sc as plsc

_B, _C, _Z, _H, _W = 2, 4, 64, 128, 128
_ROW = _H * _W
_NELEM = _B * _C * _Z * _H * _W
_INV = 1.0 / _NELEM

_ZS = 8
_Z0 = _Z - _ZS

_LOG2E = 1.4426950408889634
_LN2 = 0.6931471805599453

_ZB = 8


def _tc_body(pred_ref, gt_ref, w_ref, out_ref):
    g = gt_ref[...]
    acc = jnp.zeros_like(g, dtype=jnp.float32)
    for c in range(_C):
        t = pred_ref[0, c][None] * _LOG2E
        l = jnp.log2(1.0 + jnp.exp2(t))
        acc += jnp.where(g == c, w_ref[c] * (l - t), l)
    part = jnp.sum(acc) * (_LN2 * _INV)

    @pl.when((pl.program_id(0) == 0) & (pl.program_id(1) == 0))
    def _init():
        out_ref[...] = jnp.zeros_like(out_ref)

    out_ref[...] += part


def _kernel_tc(pred, gt, weights, z0):
    grid = (_B, z0 // _ZB)
    out = pl.pallas_call(
        _tc_body,
        grid=grid,
        in_specs=[
            pl.BlockSpec((1, _C, _ZB, _H, _W), lambda b, z: (b, 0, z, 0, 0)),
            pl.BlockSpec((1, _ZB, _H, _W), lambda b, z: (b, z, 0, 0)),
            pl.BlockSpec(memory_space=pltpu.SMEM),
        ],
        out_specs=pl.BlockSpec((1, 1), lambda b, z: (0, 0)),
        out_shape=jax.ShapeDtypeStruct((1, 1), jnp.float32),
    )(pred, gt, weights)
    return out[0, 0]


_NC, _NS = 2, 16
_NW = _NC * _NS
_NBZS = _B * _ZS
_WPB = _NW // _NBZS
_CPW = _C // _WPB

_P0 = 2.2132784e-05
_P1 = 0.999010209
_P2 = -0.489155782
_P3 = 0.283302384
_P4 = -0.13011793
_P5 = 0.0301022476


def _sc_body(pred_hbm, gt_hbm, w_hbm, out_hbm, pbuf, gbuf, wbuf, obuf,
             psem0, psem1, gsem):
    wid = lax.axis_index("c") * _NS + lax.axis_index("s")
    pltpu.sync_copy(w_hbm, wbuf)
    wvec = wbuf[...]
    psem = (psem0, psem1)

    bz = wid // _WPB
    cb0 = (wid % _WPB) * _CPW
    b = bz // _ZS
    z = _Z0 + (bz - b * _ZS)
    j = b * _Z + z

    def row_idx(ci):
        return (b * _C + cb0 + ci) * _Z + z

    gcopy = pltpu.async_copy(gt_hbm.at[j], gbuf, gsem)
    pcopy = [None] * _CPW
    pcopy[0] = pltpu.async_copy(pred_hbm.at[row_idx(0)], pbuf.at[0], psem[0])

    acc = jnp.zeros((16,), jnp.float32)
    for ci in range(_CPW):
        if ci + 1 < _CPW:
            pcopy[ci + 1] = pltpu.async_copy(
                pred_hbm.at[row_idx(ci + 1)], pbuf.at[(ci + 1) % 2],
                psem[(ci + 1) % 2])
        pcopy[ci].wait()
        if ci == 0:
            gcopy.wait()
        c = cb0 + ci
        w_sel = wvec[3]
        for cc in (2, 1, 0):
            w_sel = jnp.where(c == cc, wvec[cc], w_sel)
        pb = pbuf.at[ci % 2]

        def body(i, a, c=c, pwc=w_sel, pb=pb):
            x = pb[pl.ds(i * 16, 16)]
            g = gbuf[pl.ds(i * 16, 16)]
            e = jnp.exp(-jnp.abs(x))
            l = _P0 + e * (_P1 + e * (_P2 + e * (_P3 + e * (_P4 + e * _P5))))
            sp = jnp.maximum(x, 0.0) + l
            return a + jnp.where(g == c, pwc * (sp - x), sp)

        acc = lax.fori_loop(0, _ROW // 16, body, acc, unroll=16)
    obuf[...] = acc * _INV
    pltpu.sync_copy(obuf, out_hbm.at[wid])


def _kernel_sc(pred, gt, weights):
    p2 = pred.reshape(_B * _C * _Z, _ROW)
    g2 = gt.reshape(_B * _Z, _ROW)
    wpad = jnp.pad(weights, (0, 16 - _C))
    mesh = plsc.VectorSubcoreMesh(core_axis_name="c", subcore_axis_name="s")
    call = pl.kernel(
        _sc_body,
        out_type=jax.ShapeDtypeStruct((_NW, 16), jnp.float32),
        mesh=mesh,
        scratch_types=[
            pltpu.VMEM((2, _ROW), jnp.float32),
            pltpu.VMEM((_ROW,), jnp.int32),
            pltpu.VMEM((16,), jnp.float32),
            pltpu.VMEM((16,), jnp.float32),
            pltpu.SemaphoreType.DMA,
            pltpu.SemaphoreType.DMA,
            pltpu.SemaphoreType.DMA,
        ],
    )
    return jnp.sum(call(p2, g2, wpad))


def kernel(pred, gt, weights):
    sc_part = _kernel_sc(pred, gt, weights)
    tc_part = _kernel_tc(pred, gt, weights, _Z0)
    return tc_part + sc_part


# speedup vs baseline: 3.9369x; 1.8475x over previous
import functools

import jax
import jax.numpy as jnp
from jax import lax
from jax.experimental import pallas as pl
from jax.experimental.pallas import tpu as pltpu
from jax.experimental.pallas import tpu_sc as plsc

_B, _C, _Z, _H, _W = 2, 4, 64, 128, 128
_ROW = _H * _W
_NELEM = _B * _C * _Z * _H * _W
_INV = 1.0 / _NELEM

_ZS = 8
_Z0 = _Z - _ZS

_LOG2E = 1.4426950408889634
_LN2 = 0.6931471805599453

_ZB = 8


def _tc_body(pred_ref, gt_ref, w_ref, out_ref):
    g = gt_ref[...]
    acc = jnp.zeros_like(g, dtype=jnp.float32)
    for c in range(_C):
        t = pred_ref[0, c][None] * _LOG2E
        l = jnp.log2(1.0 + jnp.exp2(t))
        acc += jnp.where(g == c, w_ref[c] * (l - t), l)
    part = jnp.sum(acc) * (_LN2 * _INV)

    @pl.when((pl.program_id(0) == 0) & (pl.program_id(1) == 0))
    def _init():
        out_ref[...] = jnp.zeros_like(out_ref)

    out_ref[...] += part


def _kernel_tc(pred, gt, weights, z0):
    grid = (_B, z0 // _ZB)
    out = pl.pallas_call(
        _tc_body,
        grid=grid,
        in_specs=[
            pl.BlockSpec((1, _C, _ZB, _H, _W), lambda b, z: (b, 0, z, 0, 0)),
            pl.BlockSpec((1, _ZB, _H, _W), lambda b, z: (b, z, 0, 0)),
            pl.BlockSpec(memory_space=pltpu.SMEM),
        ],
        out_specs=pl.BlockSpec((1, 1), lambda b, z: (0, 0)),
        out_shape=jax.ShapeDtypeStruct((1, 1), jnp.float32),
    )(pred, gt, weights)
    return out[0, 0]


_NC, _NS = 2, 16
_NW = _NC * _NS
_NBZS = _B * _ZS
_WPB = _NW // _NBZS
_CPW = _C // _WPB

_P0 = 2.2132784e-05
_P1 = 0.999010209
_P2 = -0.489155782
_P3 = 0.283302384
_P4 = -0.13011793
_P5 = 0.0301022476


def _sc_body(pred_hbm, gt_hbm, w_hbm, out_hbm, pbuf, gbuf, wbuf, obuf,
             psem0, psem1, gsem):
    wid = lax.axis_index("c") * _NS + lax.axis_index("s")
    pltpu.sync_copy(w_hbm, wbuf)
    wvec = wbuf[...]
    psem = (psem0, psem1)

    bz = wid // _WPB
    cb0 = (wid % _WPB) * _CPW
    b = bz // _ZS
    z = _Z0 + (bz - b * _ZS)

    gcopy = pltpu.async_copy(gt_hbm.at[b, z], gbuf, gsem)
    pcopy = [None] * _CPW
    pcopy[0] = pltpu.async_copy(pred_hbm.at[b, cb0, z], pbuf.at[0], psem[0])

    acc = jnp.zeros((16,), jnp.float32)
    for ci in range(_CPW):
        if ci + 1 < _CPW:
            pcopy[ci + 1] = pltpu.async_copy(
                pred_hbm.at[b, cb0 + ci + 1, z], pbuf.at[(ci + 1) % 2],
                psem[(ci + 1) % 2])
        pcopy[ci].wait()
        if ci == 0:
            gcopy.wait()
        c = cb0 + ci
        w_sel = wvec[3]
        for cc in (2, 1, 0):
            w_sel = jnp.where(c == cc, wvec[cc], w_sel)
        pb = pbuf.at[ci % 2]

        def body(i, a, c=c, pwc=w_sel, pb=pb):
            r = i // 8
            col = (i - r * 8) * 16
            x = pb[r, pl.ds(col, 16)]
            g = gbuf[r, pl.ds(col, 16)]
            e = jnp.exp(-jnp.abs(x))
            l = _P0 + e * (_P1 + e * (_P2 + e * (_P3 + e * (_P4 + e * _P5))))
            sp = jnp.maximum(x, 0.0) + l
            return a + jnp.where(g == c, pwc * (sp - x), sp)

        acc = lax.fori_loop(0, _ROW // 16, body, acc, unroll=16)
    obuf[...] = acc * _INV
    pltpu.sync_copy(obuf, out_hbm.at[wid])


def _kernel_sc(pred, gt, weights):
    wpad = jnp.pad(weights, (0, 16 - _C))
    mesh = plsc.VectorSubcoreMesh(core_axis_name="c", subcore_axis_name="s")
    call = pl.kernel(
        _sc_body,
        out_type=jax.ShapeDtypeStruct((_NW, 16), jnp.float32),
        mesh=mesh,
        scratch_types=[
            pltpu.VMEM((2, _H, _W), jnp.float32),
            pltpu.VMEM((_H, _W), jnp.int32),
            pltpu.VMEM((16,), jnp.float32),
            pltpu.VMEM((16,), jnp.float32),
            pltpu.SemaphoreType.DMA,
            pltpu.SemaphoreType.DMA,
            pltpu.SemaphoreType.DMA,
        ],
    )
    return jnp.sum(call(pred, gt, wpad))


def kernel(pred, gt, weights):
    sc_part = _kernel_sc(pred, gt, weights)
    tc_part = _kernel_tc(pred, gt, weights, _Z0)
    return tc_part + sc_part
